# baseline (speedup 1.0000x reference)
import functools
import numpy as np
import jax
from jax import lax
import jax.numpy as jnp
from jax.experimental import pallas as pl
from jax.experimental.pallas import tpu as pltpu
from jax.experimental.pallas import tpu_sc as plsc

_NC, _NS = 2, 16


def _masked_idx(c: int, ratio: float) -> list:
    mask = np.linspace(0, c * (1 - ratio), int(c * ratio))
    return [i for i in range(c) if i in mask]


def kernel(input1, noise):
    b, c, h, w = input1.shape
    hw = h * w
    idx = _masked_idx(c, 0.5)
    nmask = len(idx)
    cb = c // nmask
    if idx != [j * cb for j in range(nmask)]:
        raise ValueError("masked channels not uniformly spaced")
    nw = _NC * _NS
    bpw = b // nw

    mesh = plsc.VectorSubcoreMesh(
        core_axis_name="c", subcore_axis_name="s",
        num_cores=_NC, num_subcores=_NS,
    )

    @functools.partial(
        pl.kernel,
        mesh=mesh,
        out_type=jax.ShapeDtypeStruct((b, c, hw), jnp.float32),
    )
    def sc_copy(x_hbm, n_hbm, o_hbm):
        wid = lax.axis_index("s") * _NC + lax.axis_index("c")
        base = wid * bpw
        pltpu.sync_copy(x_hbm.at[pl.ds(base, bpw)], o_hbm.at[pl.ds(base, bpw)])
        for bb in range(bpw):
            for j in range(nmask):
                pltpu.sync_copy(
                    n_hbm.at[j, base + bb, :],
                    o_hbm.at[base + bb, j * cb, :],
                )

    x = input1.reshape(b, c, hw)
    out = sc_copy(x, noise)
    return out.reshape(b, c, h, w)


# speedup vs baseline: 12.7522x; 12.7522x over previous
import functools
import numpy as np
import jax
from jax import lax
import jax.numpy as jnp
from jax.experimental import pallas as pl
from jax.experimental.pallas import tpu as pltpu
from jax.experimental.pallas import tpu_sc as plsc

_NC, _NS = 2, 16
_CH = 64
_NBUF = 3


def _masked_idx(c: int, ratio: float) -> list:
    mask = np.linspace(0, c * (1 - ratio), int(c * ratio))
    return [i for i in range(c) if i in mask]


def kernel(input1, noise):
    b, c, h, w = input1.shape
    hw = h * w
    idx = _masked_idx(c, 0.5)
    nmask = len(idx)
    cb = c // nmask
    if idx != [j * cb for j in range(nmask)]:
        raise ValueError("masked channels not uniformly spaced")
    nw = _NC * _NS
    bpw = b // nw
    cpb = c // _CH
    nch = bpw * cpb

    mesh = plsc.VectorSubcoreMesh(
        core_axis_name="c", subcore_axis_name="s",
        num_cores=_NC, num_subcores=_NS,
    )

    @functools.partial(
        pl.kernel,
        mesh=mesh,
        out_type=jax.ShapeDtypeStruct((b, c, hw), jnp.float32),
        scratch_types=(
            [pltpu.VMEM((_CH, hw), jnp.float32) for _ in range(_NBUF)]
            + [pltpu.SemaphoreType.DMA for _ in range(2 * _NBUF)]
        ),
    )
    def sc_copy(x_hbm, n_hbm, o_hbm, *rest):
        bufs = rest[:_NBUF]
        in_sems = rest[_NBUF:2 * _NBUF]
        out_sems = rest[2 * _NBUF:]
        wid = lax.axis_index("s") * _NC + lax.axis_index("c")
        base = wid * bpw

        def chunk(k):
            bb = base + k // cpb
            cc = (k % cpb) * _CH
            return bb, cc

        def start_in(k):
            bb, cc = chunk(k)
            return pltpu.async_copy(
                x_hbm.at[bb, pl.ds(cc, _CH), :], bufs[k % _NBUF],
                in_sems[k % _NBUF],
            )

        def start_out(k):
            bb, cc = chunk(k)
            return pltpu.async_copy(
                bufs[k % _NBUF], o_hbm.at[bb, pl.ds(cc, _CH), :],
                out_sems[k % _NBUF],
            )

        in_cps = {}
        out_cps = {}
        for k in range(min(_NBUF, nch)):
            in_cps[k] = start_in(k)
        for k in range(nch):
            in_cps.pop(k).wait()
            bb, cc = chunk(k)
            for j in range(nmask):
                if cc == j * cb:
                    pltpu.sync_copy(n_hbm.at[j, bb, :], bufs[k % _NBUF].at[0, :])
            out_cps[k] = start_out(k)
            nxt = k + _NBUF
            if nxt < nch:
                out_cps.pop(k).wait()
                in_cps[nxt] = start_in(nxt)
        for _, cp in sorted(out_cps.items()):
            cp.wait()

    x = input1.reshape(b, c, hw)
    out = sc_copy(x, noise)
    return out.reshape(b, c, h, w)


# speedup vs baseline: 45.7375x; 3.5866x over previous
import numpy as np
import jax
from jax import lax
import jax.numpy as jnp
from jax.experimental import pallas as pl


def _masked_idx(c: int, ratio: float) -> list:
    mask = np.linspace(0, c * (1 - ratio), int(c * ratio))
    return [i for i in range(c) if i in mask]


def _make_body(cb, nmask):
    def _body(x_ref, n_ref, o_ref):
        x = x_ref[...]
        lane = lax.broadcasted_iota(jnp.int32, x.shape, 3)
        r = x
        for j in range(nmask):
            nj = n_ref[:, j, :, :][..., None]
            r = jnp.where(lane == j * cb, nj, r)
        o_ref[...] = r
    return _body


def kernel(input1, noise):
    b, c, h, w = input1.shape
    idx = _masked_idx(c, 0.5)
    nmask = len(idx)
    cb = c // nmask
    if idx != [j * cb for j in range(nmask)]:
        raise ValueError("masked channels not uniformly spaced")

    xt = jnp.transpose(input1, (0, 2, 3, 1))
    nz = jnp.transpose(noise, (1, 0, 2)).reshape(b, nmask, h, w)

    out_t = pl.pallas_call(
        _make_body(cb, nmask),
        grid=(b,),
        in_specs=[
            pl.BlockSpec((1, h, w, c), lambda i: (i, 0, 0, 0)),
            pl.BlockSpec((1, nmask, h, w), lambda i: (i, 0, 0, 0)),
        ],
        out_specs=pl.BlockSpec((1, h, w, c), lambda i: (i, 0, 0, 0)),
        out_shape=jax.ShapeDtypeStruct((b, h, w, c), jnp.float32),
    )(xt, nz)
    return jnp.transpose(out_t, (0, 3, 1, 2))


# speedup vs baseline: 52.8881x; 1.1563x over previous
import numpy as np
import jax
from jax import lax
import jax.numpy as jnp
from jax.experimental import pallas as pl


def _masked_idx(c: int, ratio: float) -> list:
    mask = np.linspace(0, c * (1 - ratio), int(c * ratio))
    return [i for i in range(c) if i in mask]


def _make_body(cb, nmask):
    def _body(x_ref, n_ref, o_ref):
        x = x_ref[...]
        lane = lax.broadcasted_iota(jnp.int32, x.shape, 3)
        r = x
        for j in range(nmask):
            nj = n_ref[:, j, :, :][..., None]
            r = jnp.where(lane == j * cb, nj, r)
        o_ref[...] = r
    return _body


def kernel(input1, noise):
    b, c, h, w = input1.shape
    idx = _masked_idx(c, 0.5)
    nmask = len(idx)
    cb = c // nmask
    if idx != [j * cb for j in range(nmask)]:
        raise ValueError("masked channels not uniformly spaced")

    xt = jnp.transpose(input1, (0, 2, 3, 1))
    nz = jnp.transpose(noise, (1, 0, 2)).reshape(b, nmask, h, w)

    bb = 2
    out_t = pl.pallas_call(
        _make_body(cb, nmask),
        grid=(b // bb,),
        in_specs=[
            pl.BlockSpec((bb, h, w, c), lambda i: (i, 0, 0, 0)),
            pl.BlockSpec((bb, nmask, h, w), lambda i: (i, 0, 0, 0)),
        ],
        out_specs=pl.BlockSpec((bb, h, w, c), lambda i: (i, 0, 0, 0)),
        out_shape=jax.ShapeDtypeStruct((b, h, w, c), jnp.float32),
    )(xt, nz)
    return jnp.transpose(out_t, (0, 3, 1, 2))


# speedup vs baseline: 54.4977x; 1.0304x over previous
import numpy as np
import jax
from jax import lax
import jax.numpy as jnp
from jax.experimental import pallas as pl


def _masked_idx(c: int, ratio: float) -> list:
    mask = np.linspace(0, c * (1 - ratio), int(c * ratio))
    return [i for i in range(c) if i in mask]


def _make_body(cb, nmask):
    def _body(x_ref, n_ref, o_ref):
        x = x_ref[...]
        lane = lax.broadcasted_iota(jnp.int32, x.shape, 3)
        r = x
        for j in range(nmask):
            nj = n_ref[:, j, :, :][..., None]
            r = jnp.where(lane == j * cb, nj, r)
        o_ref[...] = r
    return _body


def kernel(input1, noise):
    b, c, h, w = input1.shape
    idx = _masked_idx(c, 0.5)
    nmask = len(idx)
    cb = c // nmask
    if idx != [j * cb for j in range(nmask)]:
        raise ValueError("masked channels not uniformly spaced")

    xt = jnp.transpose(input1, (0, 2, 3, 1))
    nz = jnp.transpose(noise, (1, 0, 2)).reshape(b, nmask, h, w)

    bb = 4
    out_t = pl.pallas_call(
        _make_body(cb, nmask),
        grid=(b // bb,),
        in_specs=[
            pl.BlockSpec((bb, h, w, c), lambda i: (i, 0, 0, 0)),
            pl.BlockSpec((bb, nmask, h, w), lambda i: (i, 0, 0, 0)),
        ],
        out_specs=pl.BlockSpec((bb, h, w, c), lambda i: (i, 0, 0, 0)),
        out_shape=jax.ShapeDtypeStruct((b, h, w, c), jnp.float32),
    )(xt, nz)
    return jnp.transpose(out_t, (0, 3, 1, 2))


# speedup vs baseline: 55.0768x; 1.0106x over previous
import numpy as np
import jax
from jax import lax
import jax.numpy as jnp
from jax.experimental import pallas as pl


def _masked_idx(c: int, ratio: float) -> list:
    mask = np.linspace(0, c * (1 - ratio), int(c * ratio))
    return [i for i in range(c) if i in mask]


def _make_body(cb, nmask):
    def _body(x_ref, n_ref, o_ref):
        x = x_ref[...]
        lane = lax.broadcasted_iota(jnp.int32, x.shape, 3)
        r = x
        for j in range(nmask):
            nj = n_ref[:, j, :, :][..., None]
            r = jnp.where(lane == j * cb, nj, r)
        o_ref[...] = r
    return _body


def kernel(input1, noise):
    b, c, h, w = input1.shape
    idx = _masked_idx(c, 0.5)
    nmask = len(idx)
    cb = c // nmask
    if idx != [j * cb for j in range(nmask)]:
        raise ValueError("masked channels not uniformly spaced")

    xt = jnp.transpose(input1, (0, 2, 3, 1))
    nz = jnp.transpose(noise, (1, 0, 2)).reshape(b, nmask, h, w)

    bb = 8
    out_t = pl.pallas_call(
        _make_body(cb, nmask),
        grid=(b // bb,),
        in_specs=[
            pl.BlockSpec((bb, h, w, c), lambda i: (i, 0, 0, 0)),
            pl.BlockSpec((bb, nmask, h, w), lambda i: (i, 0, 0, 0)),
        ],
        out_specs=pl.BlockSpec((bb, h, w, c), lambda i: (i, 0, 0, 0)),
        out_shape=jax.ShapeDtypeStruct((b, h, w, c), jnp.float32),
    )(xt, nz)
    return jnp.transpose(out_t, (0, 3, 1, 2))


# speedup vs baseline: 55.1143x; 1.0007x over previous
import numpy as np
import jax
from jax import lax
import jax.numpy as jnp
from jax.experimental import pallas as pl
from jax.experimental.pallas import tpu as pltpu


def _masked_idx(c: int, ratio: float) -> list:
    mask = np.linspace(0, c * (1 - ratio), int(c * ratio))
    return [i for i in range(c) if i in mask]


def _make_body(cb, nmask):
    def _body(x_ref, n_ref, o_ref):
        x = x_ref[...]
        lane = lax.broadcasted_iota(jnp.int32, x.shape, 3)
        r = x
        for j in range(nmask):
            nj = n_ref[:, j, :, :][..., None]
            r = jnp.where(lane == j * cb, nj, r)
        o_ref[...] = r
    return _body


def kernel(input1, noise):
    b, c, h, w = input1.shape
    idx = _masked_idx(c, 0.5)
    nmask = len(idx)
    cb = c // nmask
    if idx != [j * cb for j in range(nmask)]:
        raise ValueError("masked channels not uniformly spaced")

    xt = jnp.transpose(input1, (0, 2, 3, 1))
    nz = jnp.transpose(noise, (1, 0, 2)).reshape(b, nmask, h, w)

    bb = 8
    out_t = pl.pallas_call(
        _make_body(cb, nmask),
        grid=(b // bb,),
        in_specs=[
            pl.BlockSpec((bb, h, w, c), lambda i: (i, 0, 0, 0)),
            pl.BlockSpec((bb, nmask, h, w), lambda i: (i, 0, 0, 0)),
        ],
        out_specs=pl.BlockSpec((bb, h, w, c), lambda i: (i, 0, 0, 0)),
        out_shape=jax.ShapeDtypeStruct((b, h, w, c), jnp.float32),
        compiler_params=pltpu.CompilerParams(
            dimension_semantics=("parallel",)),
    )(xt, nz)
    return jnp.transpose(out_t, (0, 3, 1, 2))


# speedup vs baseline: 55.2457x; 1.0024x over previous
import numpy as np
import jax
from jax import lax
import jax.numpy as jnp
from jax.experimental import pallas as pl
from jax.experimental.pallas import tpu as pltpu


def _masked_idx(c: int, ratio: float) -> list:
    mask = np.linspace(0, c * (1 - ratio), int(c * ratio))
    return [i for i in range(c) if i in mask]


def _make_body(cb, nmask):
    def _body(x_ref, n_ref, o_ref):
        x = x_ref[...]
        lane = lax.broadcasted_iota(jnp.int32, x.shape, 3)
        r = x
        for j in range(nmask):
            nj = n_ref[j][..., None]
            r = jnp.where(lane == j * cb, nj, r)
        o_ref[...] = r
    return _body


def kernel(input1, noise):
    b, c, h, w = input1.shape
    idx = _masked_idx(c, 0.5)
    nmask = len(idx)
    cb = c // nmask
    if idx != [j * cb for j in range(nmask)]:
        raise ValueError("masked channels not uniformly spaced")

    xt = jnp.transpose(input1, (0, 2, 3, 1))
    nz = noise.reshape(nmask, b, h, w)

    bb = 8
    out_t = pl.pallas_call(
        _make_body(cb, nmask),
        grid=(b // bb,),
        in_specs=[
            pl.BlockSpec((bb, h, w, c), lambda i: (i, 0, 0, 0)),
            pl.BlockSpec((nmask, bb, h, w), lambda i: (0, i, 0, 0)),
        ],
        out_specs=pl.BlockSpec((bb, h, w, c), lambda i: (i, 0, 0, 0)),
        out_shape=jax.ShapeDtypeStruct((b, h, w, c), jnp.float32),
        compiler_params=pltpu.CompilerParams(
            dimension_semantics=("parallel",)),
    )(xt, nz)
    return jnp.transpose(out_t, (0, 3, 1, 2))
